# Initial kernel scaffold; baseline (speedup 1.0000x reference)
import jax
import jax.numpy as jnp
from jax.experimental import pallas as pl


def kernel(atomic_numbers, pos, edge_index, centers, widths, node_emb, layers, readout, atomic_e):
    raise NotImplementedError("write your pallas kernel here")


# speedup vs baseline: 1.7004x; 1.7004x over previous
import functools
import math

import jax
import jax.numpy as jnp
from jax import lax
from jax.experimental import pallas as pl
from jax.experimental.pallas import tpu as pltpu
from jax.experimental.pallas import tpu_sc as plsc

N = 50000
E = 800000
HIDDEN = 64
NUM_BASIS = 8
CUTOFF = 5.0

NP = 50176
EP = 802816
C = 128
HALF = 25000
AGGROWS = 25088
TRASH = 25024

def _silu(x):
    return x / (1.0 + jnp.exp(-x))


@functools.cache
def _make_k0():
    mesh = plsc.VectorSubcoreMesh(core_axis_name="c", subcore_axis_name="s")
    return functools.partial(
        pl.kernel, mesh=mesh,
        out_type=(jax.ShapeDtypeStruct((NP, HIDDEN), jnp.float32),
                  jax.ShapeDtypeStruct((NP, 16), jnp.float32)),
        scratch_types=[pltpu.VMEM((112,), jnp.int32),
                       pltpu.VMEM((112, HIDDEN), jnp.float32),
                       pltpu.VMEM((112, 16), jnp.float32),
                       pltpu.SemaphoreType.DMA],
        compiler_params=pltpu.CompilerParams(use_tc_tiling_on_sc=False),
    )(_k0_embed)


def _k0_embed(z_hbm, emb_hbm, ae_hbm, feats_out, ae_out, zbuf, ebuf, abuf, sem):
    wid = lax.axis_index("s") * 2 + lax.axis_index("c")
    base = wid * (NP // 32)

    def step(i, _):
        e0 = base + i * 112
        pltpu.sync_copy(z_hbm.at[pl.ds(e0, 112)], zbuf)
        pltpu.async_copy(emb_hbm.at[zbuf], ebuf, sem).wait()
        pltpu.async_copy(ae_hbm.at[zbuf], abuf, sem).wait()
        pltpu.sync_copy(ebuf, feats_out.at[pl.ds(e0, 112)])
        pltpu.sync_copy(abuf, ae_out.at[pl.ds(e0, 112)])
        return 0

    lax.fori_loop(0, NP // 32 // 112, step, 0)


@functools.cache
def _make_k1():
    mesh = plsc.VectorSubcoreMesh(core_axis_name="c", subcore_axis_name="s")
    return functools.partial(
        pl.kernel, mesh=mesh,
        out_type=jax.ShapeDtypeStruct((EP, 16), jnp.float32),
        scratch_types=[pltpu.VMEM((C,), jnp.int32),
                       pltpu.VMEM((C,), jnp.int32),
                       pltpu.VMEM((C, 16), jnp.float32),
                       pltpu.VMEM((C, 16), jnp.float32),
                       pltpu.SemaphoreType.DMA],
        compiler_params=pltpu.CompilerParams(use_tc_tiling_on_sc=False),
    )(_k1_edgevec)


def _k1_edgevec(pos_hbm, row_hbm, col_hbm, diff_out, rbuf, cbuf, pc, pr, sem):
    wid = lax.axis_index("s") * 2 + lax.axis_index("c")
    base = wid * (EP // 32)

    def step(i, _):
        e0 = base + i * C
        pltpu.sync_copy(row_hbm.at[pl.ds(e0, C)], rbuf)
        pltpu.sync_copy(col_hbm.at[pl.ds(e0, C)], cbuf)
        pltpu.async_copy(pos_hbm.at[cbuf], pc, sem).wait()
        pltpu.async_copy(pos_hbm.at[rbuf], pr, sem).wait()

        def sub(r, _):
            pc[r, pl.ds(0, 16)] = pc[r, pl.ds(0, 16)] - pr[r, pl.ds(0, 16)]
            return 0

        lax.fori_loop(0, C, sub, 0)
        pltpu.sync_copy(pc, diff_out.at[pl.ds(e0, C)])
        return 0

    lax.fori_loop(0, EP // 32 // C, step, 0)


@functools.cache
def _make_k3():
    mesh = plsc.VectorSubcoreMesh(core_axis_name="c", subcore_axis_name="s")
    return functools.partial(
        pl.kernel, mesh=mesh,
        out_type=jax.ShapeDtypeStruct((NP, HIDDEN), jnp.float32),
        scratch_types=[pltpu.VMEM((C,), jnp.int32),
                       pltpu.VMEM((C,), jnp.int32),
                       pltpu.VMEM((C,), jnp.int32),
                       pltpu.VMEM((C, HIDDEN), jnp.float32),
                       pltpu.VMEM((C, HIDDEN), jnp.float32),
                       pltpu.VMEM((16, HIDDEN), jnp.float32),
                       pltpu.VMEM_SHARED((AGGROWS, HIDDEN), jnp.float32),
                       pltpu.SemaphoreType.DMA],
        compiler_params=pltpu.CompilerParams(use_tc_tiling_on_sc=False),
    )(_k3_conv)


def _k3_conv(feats_hbm, w_hbm, row_hbm, col_hbm, agg_out,
             rbuf, cbuf, lbuf, fbuf, wbuf, zbuf, agg_sh, sem):
    core = lax.axis_index("c")
    sub = lax.axis_index("s")

    for r in range(16):
        for j in range(HIDDEN // 16):
            zbuf[r, pl.ds(j * 16, 16)] = jnp.zeros((16,), jnp.float32)

    zb = sub * (AGGROWS // 16)

    def zstep(i, _):
        pltpu.sync_copy(zbuf, agg_sh.at[pl.ds(zb + i * 16, 16)])
        return 0

    lax.fori_loop(0, AGGROWS // 16 // 16, zstep, 0)
    plsc.subcore_barrier()

    ebase = sub * (EP // 16)
    rlo = core * HALF

    def step(i, _):
        e0 = ebase + i * C
        pltpu.sync_copy(row_hbm.at[pl.ds(e0, C)], rbuf)
        pltpu.sync_copy(col_hbm.at[pl.ds(e0, C)], cbuf)
        pltpu.async_copy(feats_hbm.at[cbuf], fbuf, sem).wait()
        pltpu.sync_copy(w_hbm.at[pl.ds(e0, C)], wbuf)

        for j in range(C // 16):
            r16 = rbuf[pl.ds(j * 16, 16)]
            l = r16 - rlo
            eid = e0 + j * 16 + lax.broadcasted_iota(jnp.int32, (16,), 0)
            ok = (l >= 0) & (l < HALF) & (eid < E)
            lbuf[pl.ds(j * 16, 16)] = jnp.where(ok, l, TRASH)

        def mul(r, _):
            for j in range(HIDDEN // 16):
                s = pl.ds(j * 16, 16)
                fbuf[r, s] = fbuf[r, s] * wbuf[r, s]
            return 0

        lax.fori_loop(0, C, mul, 0)
        pltpu.sync_copy(fbuf, agg_sh.at[lbuf], add=True)
        return 0

    lax.fori_loop(0, EP // 16 // C, step, 0)
    plsc.subcore_barrier()

    rows_full = 1568

    @pl.when(sub < 15)
    def _():
        o = sub * rows_full
        pltpu.sync_copy(agg_sh.at[pl.ds(o, rows_full)],
                        agg_out.at[pl.ds(rlo + o, rows_full)])

    @pl.when(sub == 15)
    def _():
        o = 15 * rows_full
        pltpu.sync_copy(agg_sh.at[pl.ds(o, HALF - o)],
                        agg_out.at[pl.ds(rlo + o, HALF - o)])

    @pl.when((core == 1) & (sub == 15))
    def _():
        def zpad(i, _):
            pltpu.sync_copy(zbuf, agg_out.at[pl.ds(2 * HALF + i * 16, 16)])
            return 0

        lax.fori_loop(0, (NP - 2 * HALF) // 16, zpad, 0)


def _k2_body(diff_ref, cw_ref, ww_ref,
             w1a, b1a, w2a, b2a, w1b, b1b, w2b, b2b, w1c, b1c, w2c, b2c,
             o1, o2, o3):
    d = diff_ref[...]
    d2 = jnp.sum(d * d, axis=1, keepdims=True)
    ln = jnp.sqrt(d2)
    cut = 0.5 * (jnp.cos(ln * (math.pi / CUTOFF)) + 1.0)
    cut = cut * (ln < CUTOFF).astype(jnp.float32)
    cw = cw_ref[0:1, :]
    iw = jnp.maximum(ww_ref[0:1, :], 0.1)
    df = (ln - cw) / iw
    basis = jnp.exp(-0.5 * df * df) * cut
    for w1, b1, w2, b2, o in ((w1a, b1a, w2a, b2a, o1),
                              (w1b, b1b, w2b, b2b, o2),
                              (w1c, b1c, w2c, b2c, o3)):
        h = jnp.dot(basis, w1[...], preferred_element_type=jnp.float32)
        h = _silu(h + b1[0:1, :])
        o[...] = jnp.dot(h, w2[...], preferred_element_type=jnp.float32) + b2[0:1, :]


def _k2_radial(diff, cw, ww, wparams):
    CB = 2048
    grid = EP // CB
    full = lambda i: (0, 0)
    espec = pl.BlockSpec((CB, 16), lambda i: (i, 0))
    ospec = pl.BlockSpec((CB, HIDDEN), lambda i: (i, 0))
    wspecs = []
    args = []
    for (w1, b1, w2, b2) in wparams:
        wspecs += [pl.BlockSpec((8, HIDDEN), full), pl.BlockSpec((8, HIDDEN), full),
                   pl.BlockSpec((HIDDEN, HIDDEN), full), pl.BlockSpec((8, HIDDEN), full)]
        args += [w1, b1, w2, b2]
    return pl.pallas_call(
        _k2_body,
        grid=(grid,),
        in_specs=[espec, pl.BlockSpec((8, 8), full), pl.BlockSpec((8, 8), full)] + wspecs,
        out_specs=[ospec, ospec, ospec],
        out_shape=[jax.ShapeDtypeStruct((EP, HIDDEN), jnp.float32)] * 3,
    )(diff, cw, ww, *args)


def _k4_body(f_ref, a_ref, siw, sib, cpa, cpb, cpbias, uw1, ub1, uw2, ub2,
             g_ref, b_ref, out_ref):
    f = f_ref[...]
    a = a_ref[...]
    self_out = jnp.dot(f, siw[...], preferred_element_type=jnp.float32) + sib[0:1, :]
    conv = (jnp.dot(self_out, cpa[...], preferred_element_type=jnp.float32)
            + jnp.dot(a, cpb[...], preferred_element_type=jnp.float32)
            + cpbias[0:1, :])
    h = _silu(jnp.dot(conv, uw1[...], preferred_element_type=jnp.float32) + ub1[0:1, :])
    upd = jnp.dot(h, uw2[...], preferred_element_type=jnp.float32) + ub2[0:1, :]
    h2 = f + upd
    mu = jnp.mean(h2, axis=1, keepdims=True)
    var = jnp.mean((h2 - mu) ** 2, axis=1, keepdims=True)
    out_ref[...] = (h2 - mu) / jnp.sqrt(var + 1e-5) * g_ref[0:1, :] + b_ref[0:1, :]


def _k4_node(feats, agg, siw, sib, cpa, cpb, cpbias, uw1, ub1, uw2, ub2, g, b):
    NB = 512
    full = lambda i: (0, 0)
    nspec = pl.BlockSpec((NB, HIDDEN), lambda i: (i, 0))
    return pl.pallas_call(
        _k4_body,
        grid=(NP // NB,),
        in_specs=[nspec, nspec,
                  pl.BlockSpec((HIDDEN, HIDDEN), full), pl.BlockSpec((8, HIDDEN), full),
                  pl.BlockSpec((HIDDEN, HIDDEN), full), pl.BlockSpec((HIDDEN, HIDDEN), full),
                  pl.BlockSpec((8, HIDDEN), full),
                  pl.BlockSpec((HIDDEN, 2 * HIDDEN), full), pl.BlockSpec((8, 2 * HIDDEN), full),
                  pl.BlockSpec((2 * HIDDEN, HIDDEN), full), pl.BlockSpec((8, HIDDEN), full),
                  pl.BlockSpec((8, HIDDEN), full), pl.BlockSpec((8, HIDDEN), full)],
        out_specs=nspec,
        out_shape=jax.ShapeDtypeStruct((NP, HIDDEN), jnp.float32),
    )(feats, agg, siw, sib, cpa, cpb, cpbias, uw1, ub1, uw2, ub2, g, b)


def _k5_body(f_ref, ae_ref, w1, b1, w2, b2, w3, b3, out_ref):
    i = pl.program_id(0)
    f = f_ref[...]
    e = _silu(jnp.dot(f, w1[...], preferred_element_type=jnp.float32) + b1[0:1, :])
    e = _silu(jnp.dot(e, w2[...], preferred_element_type=jnp.float32) + b2[0:1, :])
    e = jnp.dot(e, w3[...], preferred_element_type=jnp.float32)
    ev = e[:, 0:1] + b3[0:1, 0:1] + ae_ref[:, 0:1]
    gid = i * 512 + lax.broadcasted_iota(jnp.int32, (512, 1), 0)
    ev = jnp.where(gid < N, ev, 0.0)
    part = jnp.sum(ev)
    acc = jnp.where(
        (lax.broadcasted_iota(jnp.int32, (8, 128), 0)
         + lax.broadcasted_iota(jnp.int32, (8, 128), 1)) == 0, part, 0.0)

    @pl.when(i == 0)
    def _():
        out_ref[...] = jnp.zeros((8, 128), jnp.float32)

    out_ref[...] += acc


def _k5_readout(feats, ae_g, w1, b1, w2, b2, w3, b3):
    full = lambda i: (0, 0)
    return pl.pallas_call(
        _k5_body,
        grid=(NP // 512,),
        in_specs=[pl.BlockSpec((512, HIDDEN), lambda i: (i, 0)),
                  pl.BlockSpec((512, 16), lambda i: (i, 0)),
                  pl.BlockSpec((HIDDEN, HIDDEN), full), pl.BlockSpec((8, HIDDEN), full),
                  pl.BlockSpec((HIDDEN, 32), full), pl.BlockSpec((8, 32), full),
                  pl.BlockSpec((32, 8), full), pl.BlockSpec((8, 8), full)],
        out_specs=pl.BlockSpec((8, 128), full),
        out_shape=jax.ShapeDtypeStruct((8, 128), jnp.float32),
    )(feats, ae_g, w1, b1, w2, b2, w3, b3)


def _b8(v):
    return jnp.broadcast_to(v.reshape(1, -1), (8, v.shape[-1])).astype(jnp.float32)


def kernel(atomic_numbers, pos, edge_index, centers, widths, node_emb, layers,
           readout, atomic_e):
    row = edge_index[0].astype(jnp.int32)
    col = edge_index[1].astype(jnp.int32)
    rowp = jnp.concatenate([row, jnp.zeros((EP - E,), jnp.int32)])
    colp = jnp.concatenate([col, jnp.zeros((EP - E,), jnp.int32)])
    pos16 = jnp.zeros((N, 16), jnp.float32).at[:, :3].set(pos)
    zp = jnp.concatenate([atomic_numbers.astype(jnp.int32),
                          jnp.zeros((NP - N,), jnp.int32)])
    emb_pad = jnp.zeros((128, HIDDEN), jnp.float32).at[:node_emb.shape[0]].set(node_emb)
    ae_pad = jnp.zeros((128, 16), jnp.float32).at[:atomic_e.shape[0], 0:1].set(atomic_e)

    feats, ae_g = _make_k0()(zp, emb_pad, ae_pad)
    diff = _make_k1()(pos16, rowp, colp)

    wparams = [(p['rn_W1'].T.astype(jnp.float32), _b8(p['rn_b1']),
                p['rn_W2'].T.astype(jnp.float32), _b8(p['rn_b2'])) for p in layers]
    w_all = _k2_radial(diff, _b8(centers), _b8(widths), wparams)

    for li, p in enumerate(layers):
        agg = _make_k3()(feats, w_all[li], rowp, colp)
        cpt = p['cp_W'].T
        feats = _k4_node(feats, agg,
                         p['si_W'].T.astype(jnp.float32), _b8(p['si_b']),
                         cpt[:HIDDEN].astype(jnp.float32),
                         cpt[HIDDEN:].astype(jnp.float32), _b8(p['cp_b']),
                         p['u_W1'].T.astype(jnp.float32), _b8(p['u_b1']),
                         p['u_W2'].T.astype(jnp.float32), _b8(p['u_b2']),
                         _b8(p['ln_g']), _b8(p['ln_b']))

    w3p = jnp.zeros((32, 8), jnp.float32).at[:, 0:1].set(readout['W3'].T)
    out = _k5_readout(feats, ae_g,
                      readout['W1'].T.astype(jnp.float32), _b8(readout['b1']),
                      readout['W2'].T.astype(jnp.float32), _b8(readout['b2']),
                      w3p, jnp.broadcast_to(readout['b3'].reshape(1, 1),
                                            (8, 8)).astype(jnp.float32))
    return out[0, 0]


# speedup vs baseline: 1.9002x; 1.1174x over previous
import functools
import math

import jax
import jax.numpy as jnp
from jax import lax
from jax.experimental import pallas as pl
from jax.experimental.pallas import tpu as pltpu
from jax.experimental.pallas import tpu_sc as plsc

N = 50000
E = 800000
HIDDEN = 64
NUM_BASIS = 8
CUTOFF = 5.0

NP = 50176
EP = 802816
C = 128
HALF = 25000
AGGROWS = 25088
TRASH = 25024

def _silu(x):
    return x / (1.0 + jnp.exp(-x))


@functools.cache
def _make_k0():
    mesh = plsc.VectorSubcoreMesh(core_axis_name="c", subcore_axis_name="s")
    return functools.partial(
        pl.kernel, mesh=mesh,
        out_type=(jax.ShapeDtypeStruct((NP, HIDDEN), jnp.float32),
                  jax.ShapeDtypeStruct((NP, 16), jnp.float32)),
        scratch_types=[pltpu.VMEM((112,), jnp.int32),
                       pltpu.VMEM((112, HIDDEN), jnp.float32),
                       pltpu.VMEM((112, 16), jnp.float32),
                       pltpu.SemaphoreType.DMA],
        compiler_params=pltpu.CompilerParams(use_tc_tiling_on_sc=False),
    )(_k0_embed)


def _k0_embed(z_hbm, emb_hbm, ae_hbm, feats_out, ae_out, zbuf, ebuf, abuf, sem):
    wid = lax.axis_index("s") * 2 + lax.axis_index("c")
    base = wid * (NP // 32)

    def step(i, _):
        e0 = base + i * 112
        pltpu.sync_copy(z_hbm.at[pl.ds(e0, 112)], zbuf)
        pltpu.async_copy(emb_hbm.at[zbuf], ebuf, sem).wait()
        pltpu.async_copy(ae_hbm.at[zbuf], abuf, sem).wait()
        pltpu.sync_copy(ebuf, feats_out.at[pl.ds(e0, 112)])
        pltpu.sync_copy(abuf, ae_out.at[pl.ds(e0, 112)])
        return 0

    lax.fori_loop(0, NP // 32 // 112, step, 0)


@functools.cache
def _make_k1():
    mesh = plsc.VectorSubcoreMesh(core_axis_name="c", subcore_axis_name="s")
    return functools.partial(
        pl.kernel, mesh=mesh,
        out_type=jax.ShapeDtypeStruct((EP, 16), jnp.float32),
        scratch_types=[pltpu.VMEM((C,), jnp.int32),
                       pltpu.VMEM((C,), jnp.int32),
                       pltpu.VMEM((C, 16), jnp.float32),
                       pltpu.VMEM((C, 16), jnp.float32),
                       pltpu.SemaphoreType.DMA],
        compiler_params=pltpu.CompilerParams(use_tc_tiling_on_sc=False),
    )(_k1_edgevec)


def _k1_edgevec(pos_hbm, row_hbm, col_hbm, diff_out, rbuf, cbuf, pc, pr, sem):
    wid = lax.axis_index("s") * 2 + lax.axis_index("c")
    base = wid * (EP // 32)

    def step(i, _):
        e0 = base + i * C
        pltpu.sync_copy(row_hbm.at[pl.ds(e0, C)], rbuf)
        pltpu.sync_copy(col_hbm.at[pl.ds(e0, C)], cbuf)
        pltpu.async_copy(pos_hbm.at[cbuf], pc, sem).wait()
        pltpu.async_copy(pos_hbm.at[rbuf], pr, sem).wait()

        @plsc.parallel_loop(0, C, step=1, unroll=4)
        def _(r):
            pc[r, pl.ds(0, 16)] = pc[r, pl.ds(0, 16)] - pr[r, pl.ds(0, 16)]
        pltpu.sync_copy(pc, diff_out.at[pl.ds(e0, C)])
        return 0

    lax.fori_loop(0, EP // 32 // C, step, 0)


@functools.cache
def _make_k3():
    mesh = plsc.VectorSubcoreMesh(core_axis_name="c", subcore_axis_name="s")
    return functools.partial(
        pl.kernel, mesh=mesh,
        out_type=jax.ShapeDtypeStruct((NP, HIDDEN), jnp.float32),
        scratch_types=[[pltpu.VMEM((C,), jnp.int32)] * 2,
                       [pltpu.VMEM((C,), jnp.int32)] * 2,
                       pltpu.VMEM((C,), jnp.int32),
                       [pltpu.VMEM((C, HIDDEN), jnp.float32)] * 2,
                       pltpu.VMEM((C, HIDDEN), jnp.float32),
                       pltpu.VMEM((8, HIDDEN), jnp.float32),
                       pltpu.VMEM_SHARED((AGGROWS, HIDDEN), jnp.float32),
                       [pltpu.SemaphoreType.DMA] * 2],
        compiler_params=pltpu.CompilerParams(use_tc_tiling_on_sc=False),
    )(_k3_conv)


def _k3_conv(feats_hbm, w_hbm, row_hbm, col_hbm, agg_out,
             rbufs, cbufs, lbuf, fbufs, wbuf, zbuf, agg_sh, gsems):
    core = lax.axis_index("c")
    sub = lax.axis_index("s")

    for r in range(8):
        for j in range(HIDDEN // 16):
            zbuf[r, pl.ds(j * 16, 16)] = jnp.zeros((16,), jnp.float32)

    zb = sub * (AGGROWS // 16)

    def zstep(i, _):
        pltpu.sync_copy(zbuf, agg_sh.at[pl.ds(zb + i * 8, 8)])
        return 0

    lax.fori_loop(0, AGGROWS // 16 // 8, zstep, 0)
    plsc.subcore_barrier()

    ebase = sub * (EP // 16)
    rlo = core * HALF
    STEPS = EP // 16 // C

    def issue(i, p):
        e0 = ebase + i * C
        pltpu.sync_copy(row_hbm.at[pl.ds(e0, C)], rbufs[p])
        pltpu.sync_copy(col_hbm.at[pl.ds(e0, C)], cbufs[p])
        pltpu.async_copy(feats_hbm.at[cbufs[p]], fbufs[p], gsems[p])

    def consume(i, p):
        e0 = ebase + i * C
        pltpu.make_async_copy(feats_hbm.at[cbufs[p]], fbufs[p], gsems[p]).wait()
        pltpu.sync_copy(w_hbm.at[pl.ds(e0, C)], wbuf)

        for j in range(C // 16):
            r16 = rbufs[p][pl.ds(j * 16, 16)]
            l = r16 - rlo
            eid = e0 + j * 16 + lax.broadcasted_iota(jnp.int32, (16,), 0)
            ok = (l >= 0) & (l < HALF) & (eid < E)
            lbuf[pl.ds(j * 16, 16)] = jnp.where(ok, l, TRASH)

        @plsc.parallel_loop(0, C, step=1, unroll=4)
        def _(r):
            for j in range(HIDDEN // 16):
                s = pl.ds(j * 16, 16)
                fbufs[p][r, s] = fbufs[p][r, s] * wbuf[r, s]

        pltpu.sync_copy(fbufs[p], agg_sh.at[lbuf], add=True)

    issue(0, 0)

    def pair(k, _):
        i = k * 2
        issue(i + 1, 1)
        consume(i, 0)

        @pl.when(i + 2 < STEPS)
        def _():
            issue(i + 2, 0)

        consume(i + 1, 1)
        return 0

    lax.fori_loop(0, STEPS // 2, pair, 0)
    plsc.subcore_barrier()

    rows_full = 1568

    @pl.when(sub < 15)
    def _():
        o = sub * rows_full
        pltpu.sync_copy(agg_sh.at[pl.ds(o, rows_full)],
                        agg_out.at[pl.ds(rlo + o, rows_full)])

    @pl.when(sub == 15)
    def _():
        o = 15 * rows_full
        pltpu.sync_copy(agg_sh.at[pl.ds(o, HALF - o)],
                        agg_out.at[pl.ds(rlo + o, HALF - o)])

    @pl.when((core == 1) & (sub == 15))
    def _():
        def zpad(i, _):
            pltpu.sync_copy(zbuf, agg_out.at[pl.ds(2 * HALF + i * 8, 8)])
            return 0

        lax.fori_loop(0, (NP - 2 * HALF) // 8, zpad, 0)


def _k2_body(diff_ref, cw_ref, ww_ref,
             w1a, b1a, w2a, b2a, w1b, b1b, w2b, b2b, w1c, b1c, w2c, b2c,
             o1, o2, o3):
    d = diff_ref[...]
    d2 = jnp.sum(d * d, axis=1, keepdims=True)
    ln = jnp.sqrt(d2)
    cut = 0.5 * (jnp.cos(ln * (math.pi / CUTOFF)) + 1.0)
    cut = cut * (ln < CUTOFF).astype(jnp.float32)
    cw = cw_ref[0:1, :]
    iw = jnp.maximum(ww_ref[0:1, :], 0.1)
    df = (ln - cw) / iw
    basis = jnp.exp(-0.5 * df * df) * cut
    for w1, b1, w2, b2, o in ((w1a, b1a, w2a, b2a, o1),
                              (w1b, b1b, w2b, b2b, o2),
                              (w1c, b1c, w2c, b2c, o3)):
        h = jnp.dot(basis, w1[...], preferred_element_type=jnp.float32)
        h = _silu(h + b1[0:1, :])
        o[...] = jnp.dot(h, w2[...], preferred_element_type=jnp.float32) + b2[0:1, :]


def _k2_radial(diff, cw, ww, wparams):
    CB = 2048
    grid = EP // CB
    full = lambda i: (0, 0)
    espec = pl.BlockSpec((CB, 16), lambda i: (i, 0))
    ospec = pl.BlockSpec((CB, HIDDEN), lambda i: (i, 0))
    wspecs = []
    args = []
    for (w1, b1, w2, b2) in wparams:
        wspecs += [pl.BlockSpec((8, HIDDEN), full), pl.BlockSpec((8, HIDDEN), full),
                   pl.BlockSpec((HIDDEN, HIDDEN), full), pl.BlockSpec((8, HIDDEN), full)]
        args += [w1, b1, w2, b2]
    return pl.pallas_call(
        _k2_body,
        grid=(grid,),
        in_specs=[espec, pl.BlockSpec((8, 8), full), pl.BlockSpec((8, 8), full)] + wspecs,
        out_specs=[ospec, ospec, ospec],
        out_shape=[jax.ShapeDtypeStruct((EP, HIDDEN), jnp.float32)] * 3,
    )(diff, cw, ww, *args)


def _k4_body(f_ref, a_ref, siw, sib, cpa, cpb, cpbias, uw1, ub1, uw2, ub2,
             g_ref, b_ref, out_ref):
    f = f_ref[...]
    a = a_ref[...]
    self_out = jnp.dot(f, siw[...], preferred_element_type=jnp.float32) + sib[0:1, :]
    conv = (jnp.dot(self_out, cpa[...], preferred_element_type=jnp.float32)
            + jnp.dot(a, cpb[...], preferred_element_type=jnp.float32)
            + cpbias[0:1, :])
    h = _silu(jnp.dot(conv, uw1[...], preferred_element_type=jnp.float32) + ub1[0:1, :])
    upd = jnp.dot(h, uw2[...], preferred_element_type=jnp.float32) + ub2[0:1, :]
    h2 = f + upd
    mu = jnp.mean(h2, axis=1, keepdims=True)
    var = jnp.mean((h2 - mu) ** 2, axis=1, keepdims=True)
    out_ref[...] = (h2 - mu) / jnp.sqrt(var + 1e-5) * g_ref[0:1, :] + b_ref[0:1, :]


def _k4_node(feats, agg, siw, sib, cpa, cpb, cpbias, uw1, ub1, uw2, ub2, g, b):
    NB = 512
    full = lambda i: (0, 0)
    nspec = pl.BlockSpec((NB, HIDDEN), lambda i: (i, 0))
    return pl.pallas_call(
        _k4_body,
        grid=(NP // NB,),
        in_specs=[nspec, nspec,
                  pl.BlockSpec((HIDDEN, HIDDEN), full), pl.BlockSpec((8, HIDDEN), full),
                  pl.BlockSpec((HIDDEN, HIDDEN), full), pl.BlockSpec((HIDDEN, HIDDEN), full),
                  pl.BlockSpec((8, HIDDEN), full),
                  pl.BlockSpec((HIDDEN, 2 * HIDDEN), full), pl.BlockSpec((8, 2 * HIDDEN), full),
                  pl.BlockSpec((2 * HIDDEN, HIDDEN), full), pl.BlockSpec((8, HIDDEN), full),
                  pl.BlockSpec((8, HIDDEN), full), pl.BlockSpec((8, HIDDEN), full)],
        out_specs=nspec,
        out_shape=jax.ShapeDtypeStruct((NP, HIDDEN), jnp.float32),
    )(feats, agg, siw, sib, cpa, cpb, cpbias, uw1, ub1, uw2, ub2, g, b)


def _k5_body(f_ref, ae_ref, w1, b1, w2, b2, w3, b3, out_ref):
    i = pl.program_id(0)
    f = f_ref[...]
    e = _silu(jnp.dot(f, w1[...], preferred_element_type=jnp.float32) + b1[0:1, :])
    e = _silu(jnp.dot(e, w2[...], preferred_element_type=jnp.float32) + b2[0:1, :])
    e = jnp.dot(e, w3[...], preferred_element_type=jnp.float32)
    ev = e[:, 0:1] + b3[0:1, 0:1] + ae_ref[:, 0:1]
    gid = i * 512 + lax.broadcasted_iota(jnp.int32, (512, 1), 0)
    ev = jnp.where(gid < N, ev, 0.0)
    part = jnp.sum(ev)
    acc = jnp.where(
        (lax.broadcasted_iota(jnp.int32, (8, 128), 0)
         + lax.broadcasted_iota(jnp.int32, (8, 128), 1)) == 0, part, 0.0)

    @pl.when(i == 0)
    def _():
        out_ref[...] = jnp.zeros((8, 128), jnp.float32)

    out_ref[...] += acc


def _k5_readout(feats, ae_g, w1, b1, w2, b2, w3, b3):
    full = lambda i: (0, 0)
    return pl.pallas_call(
        _k5_body,
        grid=(NP // 512,),
        in_specs=[pl.BlockSpec((512, HIDDEN), lambda i: (i, 0)),
                  pl.BlockSpec((512, 16), lambda i: (i, 0)),
                  pl.BlockSpec((HIDDEN, HIDDEN), full), pl.BlockSpec((8, HIDDEN), full),
                  pl.BlockSpec((HIDDEN, 32), full), pl.BlockSpec((8, 32), full),
                  pl.BlockSpec((32, 8), full), pl.BlockSpec((8, 8), full)],
        out_specs=pl.BlockSpec((8, 128), full),
        out_shape=jax.ShapeDtypeStruct((8, 128), jnp.float32),
    )(feats, ae_g, w1, b1, w2, b2, w3, b3)


def _b8(v):
    return jnp.broadcast_to(v.reshape(1, -1), (8, v.shape[-1])).astype(jnp.float32)


def kernel(atomic_numbers, pos, edge_index, centers, widths, node_emb, layers,
           readout, atomic_e):
    row = edge_index[0].astype(jnp.int32)
    col = edge_index[1].astype(jnp.int32)
    rowp = jnp.concatenate([row, jnp.zeros((EP - E,), jnp.int32)])
    colp = jnp.concatenate([col, jnp.zeros((EP - E,), jnp.int32)])
    pos16 = jnp.zeros((N, 16), jnp.float32).at[:, :3].set(pos)
    zp = jnp.concatenate([atomic_numbers.astype(jnp.int32),
                          jnp.zeros((NP - N,), jnp.int32)])
    emb_pad = jnp.zeros((128, HIDDEN), jnp.float32).at[:node_emb.shape[0]].set(node_emb)
    ae_pad = jnp.zeros((128, 16), jnp.float32).at[:atomic_e.shape[0], 0:1].set(atomic_e)

    feats, ae_g = _make_k0()(zp, emb_pad, ae_pad)
    diff = _make_k1()(pos16, rowp, colp)

    wparams = [(p['rn_W1'].T.astype(jnp.float32), _b8(p['rn_b1']),
                p['rn_W2'].T.astype(jnp.float32), _b8(p['rn_b2'])) for p in layers]
    w_all = _k2_radial(diff, _b8(centers), _b8(widths), wparams)

    for li, p in enumerate(layers):
        agg = _make_k3()(feats, w_all[li], rowp, colp)
        cpt = p['cp_W'].T
        feats = _k4_node(feats, agg,
                         p['si_W'].T.astype(jnp.float32), _b8(p['si_b']),
                         cpt[:HIDDEN].astype(jnp.float32),
                         cpt[HIDDEN:].astype(jnp.float32), _b8(p['cp_b']),
                         p['u_W1'].T.astype(jnp.float32), _b8(p['u_b1']),
                         p['u_W2'].T.astype(jnp.float32), _b8(p['u_b2']),
                         _b8(p['ln_g']), _b8(p['ln_b']))

    w3p = jnp.zeros((32, 8), jnp.float32).at[:, 0:1].set(readout['W3'].T)
    out = _k5_readout(feats, ae_g,
                      readout['W1'].T.astype(jnp.float32), _b8(readout['b1']),
                      readout['W2'].T.astype(jnp.float32), _b8(readout['b2']),
                      w3p, jnp.broadcast_to(readout['b3'].reshape(1, 1),
                                            (8, 8)).astype(jnp.float32))
    return out[0, 0]
